# baseline (speedup 1.0000x reference)
import functools

import jax
import jax.numpy as jnp
from jax import lax
from jax.experimental import pallas as pl
from jax.experimental.pallas import tpu as pltpu
from jax.experimental.pallas import tpu_sc as plsc

D_MODEL = 64
NUM_CORES = 2
NUM_SUBCORES = 16
LANES = 16
NUM_WORKERS = NUM_CORES * NUM_SUBCORES
NBUF = 3
SKEW = 2


def _gather_splits(hist):
    splits, off = [], 0
    while hist - off > 128:
        splits.append((off, 104))
        off += 104
    splits.append((off, hist - off))
    return splits


def _fixup_padding(idx_v, buf, b, r, hist):
    offs = list(range(0, hist - LANES + 1, LANES))
    if offs[-1] != hist - LANES:
        offs.append(hist - LANES)
    for off in offs:
        v = idx_v[r, pl.ds(off, LANES)]
        m = v == 0
        nzero = plsc.all_reduce_population_count(m)

        @pl.when(nzero[0] > 0)
        def _zero_rows(m=m, off=off):
            rows = off + lax.iota(jnp.int32, LANES)
            zeros = jnp.zeros((LANES,), jnp.float32)

            def col_body(c, cc):
                cols = jnp.full((LANES,), c, jnp.int32)
                plsc.store_scatter(buf.at[b], [rows, cols], zeros, mask=m)
                return cc

            lax.fori_loop(0, D_MODEL, col_body, 0)


def _emb_body(tok_hbm, w_hbm, out_hbm, idx_v, buf, gsem, wsem, *,
              rows_per_worker, hist):
    wid = lax.axis_index("s") * NUM_CORES + lax.axis_index("c")
    row0 = wid * rows_per_worker
    splits = _gather_splits(hist)
    pltpu.sync_copy(tok_hbm.at[pl.ds(row0, rows_per_worker)], idx_v)

    def pipe_body(i, carry):
        j_w = i - NBUF

        @pl.when((j_w >= 0) & (j_w < rows_per_worker))
        def _wait_wb():
            pltpu.make_async_copy(
                buf.at[j_w % NBUF, :, pl.ds(0, D_MODEL)],
                out_hbm.at[row0 + j_w, :, pl.ds(0, D_MODEL)],
                wsem.at[j_w % NBUF],
            ).wait()

        @pl.when(i < rows_per_worker)
        def _start_gather():
            b = i % NBUF
            for off, size in splits:
                pltpu.async_copy(
                    w_hbm.at[idx_v.at[i, pl.ds(off, size)]],
                    buf.at[b, pl.ds(off, size)],
                    gsem.at[b],
                )

        j = i - SKEW

        @pl.when((j >= 0) & (j < rows_per_worker))
        def _consume():
            b = j % NBUF
            for off, size in splits:
                pltpu.make_async_copy(
                    w_hbm.at[idx_v.at[j, pl.ds(off, size)]],
                    buf.at[b, pl.ds(off, size)],
                    gsem.at[b],
                ).wait()
            _fixup_padding(idx_v, buf, b, j, hist)
            pltpu.async_copy(
                buf.at[b, :, pl.ds(0, D_MODEL)],
                out_hbm.at[row0 + j, :, pl.ds(0, D_MODEL)],
                wsem.at[b],
            )

        return carry

    lax.fori_loop(0, rows_per_worker + NBUF, pipe_body, 0)


def kernel(tokens, weight):
    batch, hist = tokens.shape
    assert batch % NUM_WORKERS == 0
    rows_per_worker = batch // NUM_WORKERS
    idx = tokens.astype(jnp.int32)

    emb = functools.partial(
        pl.kernel,
        out_type=jax.ShapeDtypeStruct((batch, hist, 128), jnp.float32),
        mesh=plsc.VectorSubcoreMesh(core_axis_name="c", subcore_axis_name="s"),
        compiler_params=pltpu.CompilerParams(
            needs_layout_passes=False, use_tc_tiling_on_sc=False
        ),
        scratch_types=[
            pltpu.VMEM((rows_per_worker, hist), jnp.int32),
            pltpu.VMEM((NBUF, hist, 128), jnp.float32),
            pltpu.SemaphoreType.DMA((NBUF,)),
            pltpu.SemaphoreType.DMA((NBUF,)),
        ],
    )(functools.partial(_emb_body, rows_per_worker=rows_per_worker,
                        hist=hist))

    w128 = jnp.pad(weight, ((0, 0), (0, 128 - D_MODEL)))
    return emb(idx, w128)[..., :D_MODEL]


# speedup vs baseline: 1.0077x; 1.0077x over previous
import functools

import jax
import jax.numpy as jnp
from jax import lax
from jax.experimental import pallas as pl
from jax.experimental.pallas import tpu as pltpu
from jax.experimental.pallas import tpu_sc as plsc

D_MODEL = 64
NUM_CORES = 2
NUM_SUBCORES = 16
LANES = 16
NUM_WORKERS = NUM_CORES * NUM_SUBCORES
NBUF = 4
SKEW = 2


def _gather_splits(hist):
    splits, off = [], 0
    while hist - off > 128:
        splits.append((off, 104))
        off += 104
    splits.append((off, hist - off))
    return splits


def _fixup_padding(idx_v, buf, b, r, hist):
    offs = list(range(0, hist - LANES + 1, LANES))
    if offs[-1] != hist - LANES:
        offs.append(hist - LANES)
    for off in offs:
        v = idx_v[r, pl.ds(off, LANES)]
        m = v == 0
        nzero = plsc.all_reduce_population_count(m)

        @pl.when(nzero[0] > 0)
        def _zero_rows(m=m, off=off):
            rows = off + lax.iota(jnp.int32, LANES)
            zeros = jnp.zeros((LANES,), jnp.float32)

            def col_body(c, cc):
                cols = jnp.full((LANES,), c, jnp.int32)
                plsc.store_scatter(buf.at[b], [rows, cols], zeros, mask=m)
                return cc

            lax.fori_loop(0, D_MODEL, col_body, 0)


def _emb_body(tok_hbm, w_hbm, out_hbm, idx_v, buf, gsem, wsem, *,
              rows_per_worker, hist):
    wid = lax.axis_index("s") * NUM_CORES + lax.axis_index("c")
    row0 = wid * rows_per_worker
    splits = _gather_splits(hist)
    pltpu.sync_copy(tok_hbm.at[pl.ds(row0, rows_per_worker)], idx_v)

    def pipe_body(i, carry):
        j_w = i - NBUF

        @pl.when((j_w >= 0) & (j_w < rows_per_worker))
        def _wait_wb():
            pltpu.make_async_copy(
                buf.at[j_w % NBUF],
                out_hbm.at[row0 + j_w, :, pl.ds(0, D_MODEL)],
                wsem.at[j_w % NBUF],
            ).wait()

        @pl.when(i < rows_per_worker)
        def _start_gather():
            b = i % NBUF
            for off, size in splits:
                pltpu.async_copy(
                    w_hbm.at[idx_v.at[i, pl.ds(off, size)]],
                    buf.at[b, pl.ds(off, size)],
                    gsem.at[b],
                )

        j = i - SKEW

        @pl.when((j >= 0) & (j < rows_per_worker))
        def _consume():
            b = j % NBUF
            for off, size in splits:
                pltpu.make_async_copy(
                    w_hbm.at[idx_v.at[j, pl.ds(off, size)]],
                    buf.at[b, pl.ds(off, size)],
                    gsem.at[b],
                ).wait()
            _fixup_padding(idx_v, buf, b, j, hist)
            pltpu.async_copy(
                buf.at[b],
                out_hbm.at[row0 + j, :, pl.ds(0, D_MODEL)],
                wsem.at[b],
            )

        return carry

    lax.fori_loop(0, rows_per_worker + NBUF, pipe_body, 0)


def kernel(tokens, weight):
    batch, hist = tokens.shape
    assert batch % NUM_WORKERS == 0
    rows_per_worker = batch // NUM_WORKERS
    idx = tokens.astype(jnp.int32)

    emb = functools.partial(
        pl.kernel,
        out_type=jax.ShapeDtypeStruct((batch, hist, 128), jnp.float32),
        mesh=plsc.VectorSubcoreMesh(core_axis_name="c", subcore_axis_name="s"),
        compiler_params=pltpu.CompilerParams(
            needs_layout_passes=False, use_tc_tiling_on_sc=False
        ),
        scratch_types=[
            pltpu.VMEM((rows_per_worker, hist), jnp.int32),
            pltpu.VMEM((NBUF, hist, D_MODEL), jnp.float32),
            pltpu.SemaphoreType.DMA((NBUF,)),
            pltpu.SemaphoreType.DMA((NBUF,)),
        ],
    )(functools.partial(_emb_body, rows_per_worker=rows_per_worker,
                        hist=hist))

    return emb(idx, weight)[..., :D_MODEL]


# speedup vs baseline: 1.0101x; 1.0025x over previous
import functools

import jax
import jax.numpy as jnp
from jax import lax
from jax.experimental import pallas as pl
from jax.experimental.pallas import tpu as pltpu
from jax.experimental.pallas import tpu_sc as plsc

D_MODEL = 64
NUM_CORES = 2
NUM_SUBCORES = 16
LANES = 16
NUM_WORKERS = NUM_CORES * NUM_SUBCORES
NBUF = 6
SKEW = 3


def _gather_splits(hist):
    splits, off = [], 0
    while hist - off > 128:
        splits.append((off, 104))
        off += 104
    splits.append((off, hist - off))
    return splits


def _fixup_padding(idx_v, buf, b, r, hist):
    offs = list(range(0, hist - LANES + 1, LANES))
    if offs[-1] != hist - LANES:
        offs.append(hist - LANES)
    for off in offs:
        v = idx_v[r, pl.ds(off, LANES)]
        m = v == 0
        nzero = plsc.all_reduce_population_count(m)

        @pl.when(nzero[0] > 0)
        def _zero_rows(m=m, off=off):
            rows = off + lax.iota(jnp.int32, LANES)
            zeros = jnp.zeros((LANES,), jnp.float32)

            def col_body(c, cc):
                cols = jnp.full((LANES,), c, jnp.int32)
                plsc.store_scatter(buf.at[b], [rows, cols], zeros, mask=m)
                return cc

            lax.fori_loop(0, D_MODEL, col_body, 0)


def _emb_body(tok_hbm, w_hbm, out_hbm, idx_v, buf, gsem, wsem, *,
              rows_per_worker, hist):
    wid = lax.axis_index("s") * NUM_CORES + lax.axis_index("c")
    row0 = wid * rows_per_worker
    splits = _gather_splits(hist)
    pltpu.sync_copy(tok_hbm.at[pl.ds(row0, rows_per_worker)], idx_v)

    def pipe_body(i, carry):
        j_w = i - NBUF

        @pl.when((j_w >= 0) & (j_w < rows_per_worker))
        def _wait_wb():
            pltpu.make_async_copy(
                buf.at[j_w % NBUF],
                out_hbm.at[row0 + j_w, :, pl.ds(0, D_MODEL)],
                wsem.at[j_w % NBUF],
            ).wait()

        @pl.when(i < rows_per_worker)
        def _start_gather():
            b = i % NBUF
            for off, size in splits:
                pltpu.async_copy(
                    w_hbm.at[idx_v.at[i, pl.ds(off, size)]],
                    buf.at[b, pl.ds(off, size)],
                    gsem.at[b],
                )

        j = i - SKEW

        @pl.when((j >= 0) & (j < rows_per_worker))
        def _consume():
            b = j % NBUF
            for off, size in splits:
                pltpu.make_async_copy(
                    w_hbm.at[idx_v.at[j, pl.ds(off, size)]],
                    buf.at[b, pl.ds(off, size)],
                    gsem.at[b],
                ).wait()
            _fixup_padding(idx_v, buf, b, j, hist)
            pltpu.async_copy(
                buf.at[b],
                out_hbm.at[row0 + j, :, pl.ds(0, D_MODEL)],
                wsem.at[b],
            )

        return carry

    lax.fori_loop(0, rows_per_worker + NBUF, pipe_body, 0)


def kernel(tokens, weight):
    batch, hist = tokens.shape
    assert batch % NUM_WORKERS == 0
    rows_per_worker = batch // NUM_WORKERS
    idx = tokens.astype(jnp.int32)

    emb = functools.partial(
        pl.kernel,
        out_type=jax.ShapeDtypeStruct((batch, hist, 128), jnp.float32),
        mesh=plsc.VectorSubcoreMesh(core_axis_name="c", subcore_axis_name="s"),
        compiler_params=pltpu.CompilerParams(
            needs_layout_passes=False, use_tc_tiling_on_sc=False
        ),
        scratch_types=[
            pltpu.VMEM((rows_per_worker, hist), jnp.int32),
            pltpu.VMEM((NBUF, hist, D_MODEL), jnp.float32),
            pltpu.SemaphoreType.DMA((NBUF,)),
            pltpu.SemaphoreType.DMA((NBUF,)),
        ],
    )(functools.partial(_emb_body, rows_per_worker=rows_per_worker,
                        hist=hist))

    return emb(idx, weight)[..., :D_MODEL]
